# Initial kernel scaffold; baseline (speedup 1.0000x reference)
import jax
import jax.numpy as jnp
from jax.experimental import pallas as pl


def kernel(nf, ei, a1, b1n, a2, b2n, W_att, att_src, att_dst, bias_att, W1, b1, W2, b2):
    raise NotImplementedError("write your pallas kernel here")


# speedup vs baseline: 8.6175x; 8.6175x over previous
import functools

import jax
import jax.numpy as jnp
from jax import lax
from jax.experimental import pallas as pl
from jax.experimental.pallas import tpu as pltpu
from jax.experimental.pallas import tpu_sc as plsc

_N = 10000
_D = 128
_H = 8
_HD = 16
_DFF = 512
_EPS = 1e-6

_NPAD = 10240
_ETOT = _N + 320000
_B = 128
_NW = 16
_EPW = 164 * _B
_EPAD = _NW * _EPW
_NH = _NPAD // 2
_NH8 = _NH // 8
_NHP = 5248
_NH8P = 656
_ARPT = _NHP // 16
_DPT2 = _NH8P // 16
_NDG = _NPAD // 8
_DRPT = _NDG // 16
_NBLK = 512
_NGRID = _NPAD // _NBLK


def _prep_body(nf_ref, a1_ref, b1n_ref, wt_ref, asf_ref, adf_ref, p_ref,
               ts_ref, ss_ref, sd_ref, gs_ref, gd_ref, gsum_ref):
    i = pl.program_id(0)
    x = nf_ref[...]
    m = jnp.mean(x, axis=1, keepdims=True)
    xc = x - m
    var = jnp.sum(xc * xc, axis=1, keepdims=True) / (_D - 1)
    h = a1_ref[...] * xc / (jnp.sqrt(var) + _EPS) + b1n_ref[...]
    xp = jnp.dot(h, wt_ref[...], preferred_element_type=jnp.float32)
    a_src = jnp.dot(xp * asf_ref[...], p_ref[...],
                    preferred_element_type=jnp.float32)
    a_dst = jnp.dot(xp * adf_ref[...], p_ref[...],
                    preferred_element_type=jnp.float32)
    ts_ref[...] = xp
    ss_ref[...] = a_src
    sd_ref[...] = a_dst

    @pl.when(i == 0)
    def _():
        gs_ref[...] = jnp.full((8, 128), -1e30, jnp.float32)
        gd_ref[...] = jnp.full((8, 128), -1e30, jnp.float32)

    gs_ref[...] = jnp.maximum(gs_ref[...], jnp.max(a_src))
    gd_ref[...] = jnp.maximum(gd_ref[...], jnp.max(a_dst))

    @pl.when(i == _NGRID - 1)
    def _():
        gsum_ref[...] = jnp.maximum(gs_ref[...] + gd_ref[...], 0.0)


def _prep_call(nfp, a1, b1n, watt_t, asf, adf, p16):
    return pl.pallas_call(
        _prep_body,
        grid=(_NGRID,),
        in_specs=[
            pl.BlockSpec((_NBLK, _D), lambda i: (i, 0)),
            pl.BlockSpec((1, _D), lambda i: (0, 0)),
            pl.BlockSpec((1, _D), lambda i: (0, 0)),
            pl.BlockSpec((_D, _D), lambda i: (0, 0)),
            pl.BlockSpec((1, _D), lambda i: (0, 0)),
            pl.BlockSpec((1, _D), lambda i: (0, 0)),
            pl.BlockSpec((_D, _D), lambda i: (0, 0)),
        ],
        out_specs=[
            pl.BlockSpec((_NBLK, _D), lambda i: (i, 0)),
            pl.BlockSpec((_NBLK, _D), lambda i: (i, 0)),
            pl.BlockSpec((_NBLK, _D), lambda i: (i, 0)),
            pl.BlockSpec((8, 128), lambda i: (0, 0)),
            pl.BlockSpec((8, 128), lambda i: (0, 0)),
            pl.BlockSpec((8, 128), lambda i: (0, 0)),
        ],
        out_shape=[
            jax.ShapeDtypeStruct((_NPAD, _D), jnp.float32),
            jax.ShapeDtypeStruct((_NPAD, _D), jnp.float32),
            jax.ShapeDtypeStruct((_NPAD, _D), jnp.float32),
            jax.ShapeDtypeStruct((8, 128), jnp.float32),
            jax.ShapeDtypeStruct((8, 128), jnp.float32),
            jax.ShapeDtypeStruct((8, 128), jnp.float32),
        ],
    )(nfp, a1, b1n, watt_t, asf, adf, p16)


def _iota16(v):
    return lax.broadcasted_iota(jnp.int32, (16,), 0) * 0 + v


def _ramp16(v):
    return lax.broadcasted_iota(jnp.int32, (16,), 0) + v


_GATHER_DNUMS = lax.GatherDimensionNumbers(
    offset_dims=(), collapsed_slice_dims=(0,), start_index_map=(0,))


def _lane_splat(vec, lane):
    idx = _iota16(lane)
    return lax.gather(vec, idx[:, None], _GATHER_DNUMS, slice_sizes=(1,),
                      mode=lax.GatherScatterMode.PROMISE_IN_BOUNDS)


_lane_splat_i32 = _lane_splat


def _sc_body(src_hbm, dst_hbm, xp_hbm, ss_hbm, sd_hbm, g_hbm,
             acc_hbm, den_hbm,
             sidx, didx, didx8, didxm, didx8m, x_v, ss_v, sd_v, w_v, m_v,
             g_v, oidx, acc_sh, den_sh, sem1, sem2, sem3):
    s = lax.axis_index("s")
    wid = s
    zero16 = jnp.zeros((16,), jnp.float32)
    f32 = jnp.float32

    pltpu.sync_copy(g_hbm.at[0, pl.ds(0, 16)], g_v)
    gvec = g_v[...]

    for ph in range(2):
        def _zm(i, _):
            m_v[i // 8, pl.ds((i % 8) * 16, 16)] = zero16
            return 0
        lax.fori_loop(0, _B * 8, _zm, 0)

        abase = s * _ARPT
        pltpu.sync_copy(m_v, acc_sh.at[pl.ds(abase, _B)])
        pltpu.sync_copy(m_v, acc_sh.at[pl.ds(abase + _B, _B)])
        pltpu.sync_copy(m_v.at[pl.ds(0, _ARPT - 2 * _B)],
                        acc_sh.at[pl.ds(abase + 2 * _B, _ARPT - 2 * _B)])
        pltpu.sync_copy(m_v.at[pl.ds(0, _DPT2)],
                        den_sh.at[pl.ds(s * _DPT2, _DPT2)])
        plsc.subcore_barrier()

        def chunk_body(k, _):
            ebase = wid * _EPW + k * _B
            pltpu.sync_copy(src_hbm.at[pl.ds(ebase, _B)], sidx)
            pltpu.sync_copy(dst_hbm.at[pl.ds(ebase, _B)], didx)
            cp1 = pltpu.async_copy(xp_hbm.at[sidx], x_v, sem1)
            cp2 = pltpu.async_copy(ss_hbm.at[sidx], ss_v, sem2)
            cp3 = pltpu.async_copy(sd_hbm.at[didx], sd_v, sem3)
            for j in range(_B // 16):
                sl16 = pl.ds(j * 16, 16)
                d8v = lax.shift_right_logical(didx[sl16], 3)
                didx8[sl16] = d8v
                lf = (didx[sl16] - ph * _NH).astype(f32)
                ind = jnp.minimum(jnp.maximum(lf + 1.0, 0.0), 1.0) * \
                    jnp.minimum(jnp.maximum(float(_NH) - lf, 0.0), 1.0)
                didxm[sl16] = (lf * ind + float(_NH) * (1.0 - ind)) \
                    .astype(jnp.int32)
                lf8 = (d8v - ph * _NH8).astype(f32)
                ind8 = jnp.minimum(jnp.maximum(lf8 + 1.0, 0.0), 1.0) * \
                    jnp.minimum(jnp.maximum(float(_NH8) - lf8, 0.0), 1.0)
                didx8m[sl16] = (lf8 * ind8 + float(_NH8) * (1.0 - ind8)) \
                    .astype(jnp.int32)
            cp1.wait()
            cp2.wait()
            cp3.wait()

            def grp_body(j, _):
                g16f = (didx[pl.ds(j * 16, 16)] & 7).astype(f32)

                def edge_body(b2, _):
                    b = j * 16 + b2
                    e = ss_v[b, pl.ds(0, 16)] + sd_v[b, pl.ds(0, 16)]
                    el = jnp.where(e >= 0, e, 0.2 * e)
                    w = jnp.exp(el - gvec)
                    gs = _lane_splat(g16f, b2)
                    for hh in range(_H):
                        wb = _lane_splat(w, hh)
                        m_v[b, pl.ds(hh * 16, 16)] = \
                            wb * x_v[b, pl.ds(hh * 16, 16)]
                    for kk in range(8):
                        ind = jnp.maximum(1.0 - jnp.abs(gs - float(kk)), 0.0)
                        w_v[b, pl.ds(kk * 16, 16)] = w * ind
                    return 0

                lax.fori_loop(0, 16, edge_body, 0)
                return 0

            lax.fori_loop(0, _B // 16, grp_body, 0)
            pltpu.sync_copy(m_v, acc_sh.at[didxm], add=True)
            pltpu.sync_copy(w_v, den_sh.at[didx8m], add=True)
            return 0

        lax.fori_loop(0, _EPW // _B, chunk_body, 0)
        plsc.subcore_barrier()

        for q in range(_NH // 16 // 64):
            rbase = s * (_NH // 16) + q * 64
            for t in range(4):
                oidx[pl.ds(t * 16, 16)] = _ramp16(ph * _NH + rbase + t * 16)
            pltpu.sync_copy(acc_sh.at[pl.ds(rbase, 64)], m_v.at[pl.ds(0, 64)])
            pltpu.async_copy(m_v.at[pl.ds(0, 64)], acc_hbm.at[oidx],
                             sem1).wait()

        @pl.when(s < _NH8 // 64)
        def _():
            for t in range(4):
                oidx[pl.ds(t * 16, 16)] = _ramp16(ph * _NH8 + s * 64 + t * 16)
            pltpu.sync_copy(den_sh.at[pl.ds(s * 64, 64)],
                            m_v.at[pl.ds(0, 64)])
            pltpu.async_copy(m_v.at[pl.ds(0, 64)], den_hbm.at[oidx],
                             sem1).wait()

        plsc.subcore_barrier()


_sc_call = functools.partial(
    pl.kernel,
    mesh=plsc.VectorSubcoreMesh(core_axis_name="c", subcore_axis_name="s",
                                num_cores=1),
    out_type=[
        pltpu.HBM((_NPAD, 128), jnp.float32),
        pltpu.HBM((_NDG, 128), jnp.float32),
    ],
    scratch_types=[
        pltpu.VMEM((_B,), jnp.int32),
        pltpu.VMEM((_B,), jnp.int32),
        pltpu.VMEM((_B,), jnp.int32),
        pltpu.VMEM((_B,), jnp.int32),
        pltpu.VMEM((_B,), jnp.int32),
        pltpu.VMEM((_B, 128), jnp.float32),
        pltpu.VMEM((_B, 128), jnp.float32),
        pltpu.VMEM((_B, 128), jnp.float32),
        pltpu.VMEM((_B, 128), jnp.float32),
        pltpu.VMEM((_B, 128), jnp.float32),
        pltpu.VMEM((16,), jnp.float32),
        pltpu.VMEM((64,), jnp.int32),
        pltpu.VMEM_SHARED((_NHP, 128), jnp.float32),
        pltpu.VMEM_SHARED((_NH8P, 128), jnp.float32),
        pltpu.SemaphoreType.DMA,
        pltpu.SemaphoreType.DMA,
        pltpu.SemaphoreType.DMA,
    ],
)(_sc_body)


def _ffn_body(nf_ref, acc_ref, den_ref, q_ref, bias_ref,
              a2_ref, b2n_ref, w1_ref, b1_ref, w2_ref, b2_ref, out_ref):
    accs = acc_ref[...]
    dens = den_ref[...]
    den_big = jnp.dot(dens, q_ref[...], preferred_element_type=jnp.float32)
    nf2 = nf_ref[...] + accs / (den_big + 1e-16) + bias_ref[...]
    m = jnp.mean(nf2, axis=1, keepdims=True)
    xc = nf2 - m
    var = jnp.sum(xc * xc, axis=1, keepdims=True) / (_D - 1)
    h2 = a2_ref[...] * xc / (jnp.sqrt(var) + _EPS) + b2n_ref[...]
    ff1 = jnp.maximum(
        jnp.dot(h2, w1_ref[...], preferred_element_type=jnp.float32)
        + b1_ref[...], 0.0)
    ff2 = jnp.dot(ff1, w2_ref[...], preferred_element_type=jnp.float32) \
        + b2_ref[...]
    out_ref[...] = nf2 + ff2


def _ffn_call(nfp, acc, den, q, bias, a2, b2n, w1t, b1, w2t, b2):
    return pl.pallas_call(
        _ffn_body,
        grid=(_NGRID,),
        in_specs=[
            pl.BlockSpec((_NBLK, _D), lambda i: (i, 0)),
            pl.BlockSpec((_NBLK, _D), lambda i: (i, 0)),
            pl.BlockSpec((_NBLK, 16), lambda i: (i, 0)),
            pl.BlockSpec((16, _D), lambda i: (0, 0)),
            pl.BlockSpec((1, _D), lambda i: (0, 0)),
            pl.BlockSpec((1, _D), lambda i: (0, 0)),
            pl.BlockSpec((1, _D), lambda i: (0, 0)),
            pl.BlockSpec((_D, _DFF), lambda i: (0, 0)),
            pl.BlockSpec((1, _DFF), lambda i: (0, 0)),
            pl.BlockSpec((_DFF, _D), lambda i: (0, 0)),
            pl.BlockSpec((1, _D), lambda i: (0, 0)),
        ],
        out_specs=pl.BlockSpec((_NBLK, _D), lambda i: (i, 0)),
        out_shape=jax.ShapeDtypeStruct((_NPAD, _D), jnp.float32),
    )(nfp, acc, den, q, bias, a2, b2n, w1t, b1, w2t, b2)


def kernel(nf, ei, a1, b1n, a2, b2n, W_att, att_src, att_dst, bias_att,
           W1, b1, W2, b2):
    f32 = jnp.float32
    nfp = jnp.pad(nf, ((0, _NPAD - _N), (0, 0)))
    loop = jnp.arange(_N, dtype=jnp.int32)
    src = jnp.concatenate([ei[0].astype(jnp.int32), loop])
    dst = jnp.concatenate([ei[1].astype(jnp.int32), loop])
    src = jnp.pad(src, (0, _EPAD - _ETOT), constant_values=_N)
    dst = jnp.pad(dst, (0, _EPAD - _ETOT), constant_values=_N)

    p128 = (jnp.arange(_D)[:, None] // _HD == jnp.arange(_D)[None, :]) \
        .astype(f32)
    q16 = (jnp.arange(16)[:, None] == jnp.arange(_D)[None, :] // _HD) \
        .astype(f32)

    ts, ss, sd, _, _, gsum = _prep_call(
        nfp, a1.reshape(1, _D), b1n.reshape(1, _D), W_att.T,
        att_src.reshape(1, _D), att_dst.reshape(1, _D), p128)

    acc, den = _sc_call(src, dst, ts, ss, sd, gsum)
    den = den.reshape(_NPAD, 16)

    out = _ffn_call(
        nfp, acc, den, q16, bias_att.reshape(1, _D),
        a2.reshape(1, _D), b2n.reshape(1, _D), W1.T, b1.reshape(1, _DFF),
        W2.T, b2.reshape(1, _D))
    return out[:_N]


# speedup vs baseline: 9.7693x; 1.1337x over previous
import functools

import jax
import jax.numpy as jnp
from jax import lax
from jax.experimental import pallas as pl
from jax.experimental.pallas import tpu as pltpu
from jax.experimental.pallas import tpu_sc as plsc

_N = 10000
_D = 128
_H = 8
_HD = 16
_DFF = 512
_EPS = 1e-6

_NPAD = 10240
_ETOT = _N + 320000
_B = 128
_NW = 16
_EPW = 164 * _B
_EPAD = _NW * _EPW
_NH = _NPAD // 2
_NH8 = _NH // 8
_NHP = 5248
_NH8P = 656
_ARPT = _NHP // 16
_DPT2 = _NH8P // 16
_NDG = _NPAD // 8
_DRPT = _NDG // 16
_NBLK = 512
_NGRID = _NPAD // _NBLK


def _prep_body(nf_ref, a1_ref, b1n_ref, wt_ref, asf_ref, adf_ref, p_ref,
               ts_ref, sd_ref, gs_ref, gd_ref, gsum_ref):
    i = pl.program_id(0)
    x = nf_ref[...]
    m = jnp.mean(x, axis=1, keepdims=True)
    xc = x - m
    var = jnp.sum(xc * xc, axis=1, keepdims=True) / (_D - 1)
    h = a1_ref[...] * xc / (jnp.sqrt(var) + _EPS) + b1n_ref[...]
    xp = jnp.dot(h, wt_ref[...], preferred_element_type=jnp.float32)
    a_src = jnp.dot(xp * asf_ref[...], p_ref[...],
                    preferred_element_type=jnp.float32)
    a_dst = jnp.dot(xp * adf_ref[...], p_ref[...],
                    preferred_element_type=jnp.float32)
    ts_ref[...] = jnp.concatenate([xp, a_src], axis=1)
    sd_ref[...] = a_dst

    @pl.when(i == 0)
    def _():
        gs_ref[...] = jnp.full((8, 128), -1e30, jnp.float32)
        gd_ref[...] = jnp.full((8, 128), -1e30, jnp.float32)

    gs_ref[...] = jnp.maximum(gs_ref[...], jnp.max(a_src))
    gd_ref[...] = jnp.maximum(gd_ref[...], jnp.max(a_dst))

    @pl.when(i == _NGRID - 1)
    def _():
        gsum_ref[...] = jnp.maximum(gs_ref[...] + gd_ref[...], 0.0)


def _prep_call(nfp, a1, b1n, watt_t, asf, adf, p16):
    return pl.pallas_call(
        _prep_body,
        grid=(_NGRID,),
        in_specs=[
            pl.BlockSpec((_NBLK, _D), lambda i: (i, 0)),
            pl.BlockSpec((1, _D), lambda i: (0, 0)),
            pl.BlockSpec((1, _D), lambda i: (0, 0)),
            pl.BlockSpec((_D, _D), lambda i: (0, 0)),
            pl.BlockSpec((1, _D), lambda i: (0, 0)),
            pl.BlockSpec((1, _D), lambda i: (0, 0)),
            pl.BlockSpec((_D, _D), lambda i: (0, 0)),
        ],
        out_specs=[
            pl.BlockSpec((_NBLK, 2 * _D), lambda i: (i, 0)),
            pl.BlockSpec((_NBLK, _D), lambda i: (i, 0)),
            pl.BlockSpec((8, 128), lambda i: (0, 0)),
            pl.BlockSpec((8, 128), lambda i: (0, 0)),
            pl.BlockSpec((8, 128), lambda i: (0, 0)),
        ],
        out_shape=[
            jax.ShapeDtypeStruct((_NPAD, 2 * _D), jnp.float32),
            jax.ShapeDtypeStruct((_NPAD, _D), jnp.float32),
            jax.ShapeDtypeStruct((8, 128), jnp.float32),
            jax.ShapeDtypeStruct((8, 128), jnp.float32),
            jax.ShapeDtypeStruct((8, 128), jnp.float32),
        ],
    )(nfp, a1, b1n, watt_t, asf, adf, p16)


def _iota16(v):
    return lax.broadcasted_iota(jnp.int32, (16,), 0) * 0 + v


def _ramp16(v):
    return lax.broadcasted_iota(jnp.int32, (16,), 0) + v


_GATHER_DNUMS = lax.GatherDimensionNumbers(
    offset_dims=(), collapsed_slice_dims=(0,), start_index_map=(0,))


def _lane_splat(vec, lane):
    idx = _iota16(lane)
    return lax.gather(vec, idx[:, None], _GATHER_DNUMS, slice_sizes=(1,),
                      mode=lax.GatherScatterMode.PROMISE_IN_BOUNDS)


_lane_splat_i32 = _lane_splat


def _sc_body(src_hbm, dst_hbm, xp_hbm, sd_hbm, g_hbm,
             acc_hbm, den_hbm,
             sidx, didx, didx8, didxm, didx8m, x_v, sd_v, w_v, m_v,
             g_v, oidx, acc_sh, den_sh, sem1, sem3):
    s = lax.axis_index("s")
    wid = s
    zero16 = jnp.zeros((16,), jnp.float32)
    f32 = jnp.float32

    pltpu.sync_copy(g_hbm.at[0, pl.ds(0, 16)], g_v)
    gvec = g_v[...]

    for ph in range(2):
        def _zm(i, _):
            m_v[i // 8, pl.ds((i % 8) * 16, 16)] = zero16
            return 0
        lax.fori_loop(0, _B * 8, _zm, 0)

        abase = s * _ARPT
        pltpu.sync_copy(m_v, acc_sh.at[pl.ds(abase, _B)])
        pltpu.sync_copy(m_v, acc_sh.at[pl.ds(abase + _B, _B)])
        pltpu.sync_copy(m_v.at[pl.ds(0, _ARPT - 2 * _B)],
                        acc_sh.at[pl.ds(abase + 2 * _B, _ARPT - 2 * _B)])
        pltpu.sync_copy(m_v.at[pl.ds(0, _DPT2)],
                        den_sh.at[pl.ds(s * _DPT2, _DPT2)])
        plsc.subcore_barrier()

        def chunk_body(k, _):
            ebase = wid * _EPW + k * _B
            pltpu.sync_copy(src_hbm.at[pl.ds(ebase, _B)], sidx)
            pltpu.sync_copy(dst_hbm.at[pl.ds(ebase, _B)], didx)
            cp1 = pltpu.async_copy(xp_hbm.at[sidx], x_v, sem1)
            cp3 = pltpu.async_copy(sd_hbm.at[didx], sd_v, sem3)
            for j in range(_B // 16):
                sl16 = pl.ds(j * 16, 16)
                d8v = lax.shift_right_logical(didx[sl16], 3)
                didx8[sl16] = d8v
                lf = (didx[sl16] - ph * _NH).astype(f32)
                ind = jnp.minimum(jnp.maximum(lf + 1.0, 0.0), 1.0) * \
                    jnp.minimum(jnp.maximum(float(_NH) - lf, 0.0), 1.0)
                didxm[sl16] = (lf * ind + float(_NH) * (1.0 - ind)) \
                    .astype(jnp.int32)
                lf8 = (d8v - ph * _NH8).astype(f32)
                ind8 = jnp.minimum(jnp.maximum(lf8 + 1.0, 0.0), 1.0) * \
                    jnp.minimum(jnp.maximum(float(_NH8) - lf8, 0.0), 1.0)
                didx8m[sl16] = (lf8 * ind8 + float(_NH8) * (1.0 - ind8)) \
                    .astype(jnp.int32)
            cp1.wait()
            cp3.wait()

            def grp_body(j, _):
                g16f = (didx[pl.ds(j * 16, 16)] & 7).astype(f32)

                def edge_body(b2, _):
                    b = j * 16 + b2
                    e = x_v[b, pl.ds(_D, 16)] + sd_v[b, pl.ds(0, 16)]
                    el = jnp.where(e >= 0, e, 0.2 * e)
                    w = jnp.exp(el - gvec)
                    gs = _lane_splat(g16f, b2)
                    for hh in range(_H):
                        wb = _lane_splat(w, hh)
                        m_v[b, pl.ds(hh * 16, 16)] = \
                            wb * x_v[b, pl.ds(hh * 16, 16)]
                    for kk in range(8):
                        ind = jnp.maximum(1.0 - jnp.abs(gs - float(kk)), 0.0)
                        w_v[b, pl.ds(kk * 16, 16)] = w * ind
                    return 0

                lax.fori_loop(0, 16, edge_body, 0)
                return 0

            lax.fori_loop(0, _B // 16, grp_body, 0)
            pltpu.sync_copy(m_v, acc_sh.at[didxm], add=True)
            pltpu.sync_copy(w_v, den_sh.at[didx8m], add=True)
            return 0

        lax.fori_loop(0, _EPW // _B, chunk_body, 0)
        plsc.subcore_barrier()

        for q in range(_NH // 16 // 64):
            rbase = s * (_NH // 16) + q * 64
            for t in range(4):
                oidx[pl.ds(t * 16, 16)] = _ramp16(ph * _NH + rbase + t * 16)
            pltpu.sync_copy(acc_sh.at[pl.ds(rbase, 64)], m_v.at[pl.ds(0, 64)])
            pltpu.async_copy(m_v.at[pl.ds(0, 64)], acc_hbm.at[oidx],
                             sem1).wait()

        @pl.when(s < _NH8 // 64)
        def _():
            for t in range(4):
                oidx[pl.ds(t * 16, 16)] = _ramp16(ph * _NH8 + s * 64 + t * 16)
            pltpu.sync_copy(den_sh.at[pl.ds(s * 64, 64)],
                            m_v.at[pl.ds(0, 64)])
            pltpu.async_copy(m_v.at[pl.ds(0, 64)], den_hbm.at[oidx],
                             sem1).wait()

        plsc.subcore_barrier()


_sc_call = functools.partial(
    pl.kernel,
    mesh=plsc.VectorSubcoreMesh(core_axis_name="c", subcore_axis_name="s",
                                num_cores=1),
    out_type=[
        pltpu.HBM((_NPAD, 128), jnp.float32),
        pltpu.HBM((_NDG, 128), jnp.float32),
    ],
    scratch_types=[
        pltpu.VMEM((_B,), jnp.int32),
        pltpu.VMEM((_B,), jnp.int32),
        pltpu.VMEM((_B,), jnp.int32),
        pltpu.VMEM((_B,), jnp.int32),
        pltpu.VMEM((_B,), jnp.int32),
        pltpu.VMEM((_B, 2 * _D), jnp.float32),
        pltpu.VMEM((_B, 128), jnp.float32),
        pltpu.VMEM((_B, 128), jnp.float32),
        pltpu.VMEM((_B, 128), jnp.float32),
        pltpu.VMEM((16,), jnp.float32),
        pltpu.VMEM((64,), jnp.int32),
        pltpu.VMEM_SHARED((_NHP, 128), jnp.float32),
        pltpu.VMEM_SHARED((_NH8P, 128), jnp.float32),
        pltpu.SemaphoreType.DMA,
        pltpu.SemaphoreType.DMA,
    ],
)(_sc_body)


def _ffn_body(nf_ref, acc_ref, den_ref, q_ref, bias_ref,
              a2_ref, b2n_ref, w1_ref, b1_ref, w2_ref, b2_ref, out_ref):
    accs = acc_ref[...]
    dens = den_ref[...]
    den_big = jnp.dot(dens, q_ref[...], preferred_element_type=jnp.float32)
    nf2 = nf_ref[...] + accs / (den_big + 1e-16) + bias_ref[...]
    m = jnp.mean(nf2, axis=1, keepdims=True)
    xc = nf2 - m
    var = jnp.sum(xc * xc, axis=1, keepdims=True) / (_D - 1)
    h2 = a2_ref[...] * xc / (jnp.sqrt(var) + _EPS) + b2n_ref[...]
    ff1 = jnp.maximum(
        jnp.dot(h2, w1_ref[...], preferred_element_type=jnp.float32)
        + b1_ref[...], 0.0)
    ff2 = jnp.dot(ff1, w2_ref[...], preferred_element_type=jnp.float32) \
        + b2_ref[...]
    out_ref[...] = nf2 + ff2


def _ffn_call(nfp, acc, den, q, bias, a2, b2n, w1t, b1, w2t, b2):
    return pl.pallas_call(
        _ffn_body,
        grid=(_NGRID,),
        in_specs=[
            pl.BlockSpec((_NBLK, _D), lambda i: (i, 0)),
            pl.BlockSpec((_NBLK, _D), lambda i: (i, 0)),
            pl.BlockSpec((_NBLK, 16), lambda i: (i, 0)),
            pl.BlockSpec((16, _D), lambda i: (0, 0)),
            pl.BlockSpec((1, _D), lambda i: (0, 0)),
            pl.BlockSpec((1, _D), lambda i: (0, 0)),
            pl.BlockSpec((1, _D), lambda i: (0, 0)),
            pl.BlockSpec((_D, _DFF), lambda i: (0, 0)),
            pl.BlockSpec((1, _DFF), lambda i: (0, 0)),
            pl.BlockSpec((_DFF, _D), lambda i: (0, 0)),
            pl.BlockSpec((1, _D), lambda i: (0, 0)),
        ],
        out_specs=pl.BlockSpec((_NBLK, _D), lambda i: (i, 0)),
        out_shape=jax.ShapeDtypeStruct((_NPAD, _D), jnp.float32),
    )(nfp, acc, den, q, bias, a2, b2n, w1t, b1, w2t, b2)


def kernel(nf, ei, a1, b1n, a2, b2n, W_att, att_src, att_dst, bias_att,
           W1, b1, W2, b2):
    f32 = jnp.float32
    nfp = jnp.pad(nf, ((0, _NPAD - _N), (0, 0)))
    loop = jnp.arange(_N, dtype=jnp.int32)
    src = jnp.concatenate([ei[0].astype(jnp.int32), loop])
    dst = jnp.concatenate([ei[1].astype(jnp.int32), loop])
    src = jnp.pad(src, (0, _EPAD - _ETOT), constant_values=_N)
    dst = jnp.pad(dst, (0, _EPAD - _ETOT), constant_values=_N)

    p128 = (jnp.arange(_D)[:, None] // _HD == jnp.arange(_D)[None, :]) \
        .astype(f32)
    q16 = (jnp.arange(16)[:, None] == jnp.arange(_D)[None, :] // _HD) \
        .astype(f32)

    ts, sd, _, _, gsum = _prep_call(
        nfp, a1.reshape(1, _D), b1n.reshape(1, _D), W_att.T,
        att_src.reshape(1, _D), att_dst.reshape(1, _D), p128)

    acc, den = _sc_call(src, dst, ts, sd, gsum)
    den = den.reshape(_NPAD, 16)

    out = _ffn_call(
        nfp, acc, den, q16, bias_att.reshape(1, _D),
        a2.reshape(1, _D), b2n.reshape(1, _D), W1.T, b1.reshape(1, _DFF),
        W2.T, b2.reshape(1, _D))
    return out[:_N]
